# baseline (speedup 1.0000x reference)
import functools

import jax
import jax.numpy as jnp
import numpy as np
from jax import lax
from jax.experimental import pallas as pl
from jax.experimental.pallas import tpu as pltpu
from jax.experimental.pallas import tpu_sc as plsc

_S = 64
_P = 128
_H = 12
_D = 64
_C = _H * _D
_N = _S * _P
_KN = 8
_G = 2
_NS = _KN + _G - 1
_NT = _S // _G
_KEYS = _NS * _P

_QSCALE = np.float32(np.log2(np.e) / np.sqrt(_D))
_LM_SHIFT = np.float32(2.0 ** -20)


def _qkv_body(x_ref, w_ref, b_ref, q_ref, kv_ref):
    y = jnp.dot(x_ref[...].astype(jnp.bfloat16),
                w_ref[...].astype(jnp.bfloat16),
                preferred_element_type=jnp.float32)
    y = y + b_ref[...]
    q_ref[...] = (y[:, :_C] * _QSCALE).astype(jnp.bfloat16)
    kv_ref[...] = y[:, _C:].astype(jnp.bfloat16)


def _attn_body(F_ref, lm_ref, q_ref, *rest):
    kv_refs = rest[0:_NS]
    wp_ref = rest[_NS]
    bp_ref = rest[_NS + 1]
    o_ref = rest[_NS + 2]
    t = pl.program_id(0)

    la = [(lm_ref[c * _NT + t].astype(jnp.float32) * _LM_SHIFT
           ).astype(jnp.bfloat16) for c in range(_NS)]
    lb = [(lm_ref[(_NS + c) * _NT + t].astype(jnp.float32) * _LM_SHIFT
           ).astype(jnp.bfloat16) for c in range(_NS)]

    zpad = jnp.zeros((_P, _D - _G), jnp.bfloat16)
    kbias = jnp.concatenate(
        [jnp.concatenate(
            [jnp.full((_P, 1), la[c], jnp.bfloat16),
             jnp.full((_P, 1), lb[c], jnp.bfloat16), zpad], axis=1)
         for c in range(_NS)], axis=0)

    row = jax.lax.broadcasted_iota(jnp.int32, (_G * _P, _D), 0)
    col = jax.lax.broadcasted_iota(jnp.int32, (_G * _P, _D), 1)
    e2 = jnp.where((col == 0) & (row < _P), 1.0,
                   jnp.where((col == 1) & (row >= _P), 1.0, 0.0)
                   ).astype(jnp.bfloat16)

    ones_col = jnp.full((_P, _D), 1.0, jnp.bfloat16)
    ones_blk = jnp.concatenate([ones_col] * _NS, axis=0)

    q = q_ref[...]
    q_parts, k_cols, v_cols = [], [], []
    for h in range(_H):
        sl = slice(h * _D, (h + 1) * _D)
        q_parts.append(q[:, sl])
        q_parts.append(e2)
        k_cols.append(jnp.concatenate(
            [kv_refs[c][0][:, sl] for c in range(_NS)], axis=0))
        k_cols.append(kbias)
        v_cols.append(jnp.concatenate(
            [kv_refs[c][0][:, _C + h * _D:_C + (h + 1) * _D]
             for c in range(_NS)], axis=0))
        v_cols.append(ones_blk)
    q_all = jnp.concatenate(q_parts, axis=1)
    k_all = jnp.concatenate(k_cols, axis=1)
    v_all = jnp.concatenate(v_cols, axis=1)

    outs = []
    for h in range(_H):
        sl2 = slice(2 * h * _D, 2 * (h + 1) * _D)
        s = jax.lax.dot_general(q_all[:, sl2], k_all[:, sl2],
                                (((1,), (1,)), ((), ())),
                                preferred_element_type=jnp.float32)
        p = jnp.exp2(s).astype(jnp.bfloat16)
        nd = jnp.dot(p, v_all[:, sl2],
                     preferred_element_type=jnp.float32)
        outs.append(nd[:, :_D] / nd[:, _D:_D + 1])
    o = jnp.concatenate(outs, axis=1).astype(jnp.bfloat16)
    o_ref[...] = jnp.dot(o, wp_ref[...],
                         preferred_element_type=jnp.float32) + bp_ref[...]


def _qkv_call(x2, W_qkv, b2):
    blk = 1024
    grid = (_N // blk,)
    shp = jax.ShapeDtypeStruct((_N, _C), jnp.bfloat16)
    return pl.pallas_call(
        _qkv_body,
        grid=grid,
        in_specs=[
            pl.BlockSpec((blk, _C), lambda i: (i, 0)),
            pl.BlockSpec((_C, 3 * _C), lambda i: (0, 0)),
            pl.BlockSpec((1, 3 * _C), lambda i: (0, 0)),
        ],
        out_specs=[
            pl.BlockSpec((blk, _C), lambda i: (i, 0)),
            pl.BlockSpec((blk, 2 * _C), lambda i: (i, 0)),
        ],
        out_shape=[shp, jax.ShapeDtypeStruct((_N, 2 * _C), jnp.bfloat16)],
    )(x2, W_qkv, b2)


def _attn_call(F_flat, lm_flat, q2, kv3, W_proj, bp2):
    kv_spec = [
        pl.BlockSpec((1, _P, 2 * _C),
                     functools.partial(lambda t, F, lm, c: (F[c * _NT + t], 0, 0),
                                       c=c))
        for c in range(_NS)
    ]
    grid_spec = pltpu.PrefetchScalarGridSpec(
        num_scalar_prefetch=2,
        grid=(_NT,),
        in_specs=[
            pl.BlockSpec((_G * _P, _C), lambda t, F, lm: (t, 0)),
            *kv_spec,
            pl.BlockSpec((_C, _C), lambda t, F, lm: (0, 0)),
            pl.BlockSpec((1, _C), lambda t, F, lm: (0, 0)),
        ],
        out_specs=pl.BlockSpec((_G * _P, _C), lambda t, F, lm: (t, 0)),
    )
    return pl.pallas_call(
        _attn_body,
        grid_spec=grid_spec,
        out_shape=jax.ShapeDtypeStruct((_N, _C), jnp.float32),
    )(F_flat, lm_flat, q2, *([kv3] * _NS), W_proj, bp2)


_LM_TABLE = [-(2 ** 30)] + [
    int(x) for x in np.round(np.log2(np.arange(1, _KN + 1)) * 2.0 ** 20)]


def _prep_sc_body(hs_ref, f_ref, lm_ref, hv, fv, lmv):
    wid = lax.axis_index("c") * 16 + lax.axis_index("s")

    @pl.when(wid == 0)
    def _():
        pltpu.sync_copy(hs_ref, hv)
        for g in range(_NT // 16):
            frames = [hv[pl.ds(j * _NT + g * 16, 16)] >> 7
                      for j in range(_G * _KN)]
            mn = frames[0]
            for j in range(1, _G * _KN):
                mn = jnp.minimum(mn, frames[j])
            su = jnp.minimum(mn, _S - _KN)
            for c in range(_NS):
                r = lax.rem(lax.rem(c - su, _NS) + _NS, _NS)
                nom = su + r
                fv[pl.ds(c * _NT + g * 16, 16)] = jnp.minimum(nom, _S - 1)
                for a in range(_G):
                    mm = jnp.zeros((16,), jnp.int32)
                    for j in range(_KN):
                        mm = mm + jnp.where(frames[a * _KN + j] == nom, 1, 0)
                    lmval = jnp.full((16,), _LM_TABLE[0], jnp.int32)
                    for mv in range(1, _KN + 1):
                        lmval = jnp.where(mm == mv, _LM_TABLE[mv], lmval)
                    lmv[pl.ds((a * _NS + c) * _NT + g * 16, 16)] = lmval
        pltpu.sync_copy(fv, f_ref)
        pltpu.sync_copy(lmv, lm_ref)


def _prep(gather_idx):
    heads = gather_idx.astype(jnp.int32).reshape(_S, _KN, _P)[:, :, 0]
    hs = heads.reshape(_NT, _G * _KN).T.reshape(-1)
    mesh = plsc.VectorSubcoreMesh(core_axis_name="c", subcore_axis_name="s")
    fn = pl.kernel(
        _prep_sc_body,
        out_type=[jax.ShapeDtypeStruct((_NS * _NT,), jnp.int32),
                  jax.ShapeDtypeStruct((_G * _NS * _NT,), jnp.int32)],
        mesh=mesh,
        scratch_types=[pltpu.VMEM((_G * _KN * _NT,), jnp.int32),
                       pltpu.VMEM((_NS * _NT,), jnp.int32),
                       pltpu.VMEM((_G * _NS * _NT,), jnp.int32)],
    )
    return fn(hs)


def kernel(x, W_qkv, b_qkv, W_proj, b_proj, gather_idx):
    B_, N_, C_ = x.shape
    x2 = x.reshape(N_, C_)
    F_flat, lm_flat = _prep(gather_idx)
    q2, kv2 = _qkv_call(x2, W_qkv, b_qkv.reshape(1, 3 * _C))
    kv3 = kv2.reshape(_S, _P, 2 * _C)
    out = _attn_call(F_flat, lm_flat, q2, kv3,
                     W_proj.astype(jnp.bfloat16), b_proj.reshape(1, _C))
    return out.reshape(B_, N_, C_)
